# baseline (speedup 1.0000x reference)
import functools

import jax
import jax.numpy as jnp
from jax import lax
from jax.experimental import pallas as pl
from jax.experimental.pallas import tpu as pltpu
from jax.experimental.pallas import tpu_sc as plsc

_NE = 10000
_NC = 2
_NS = 16
_L = 16
_C = 112
_NB = 3
_PB = 14


def _sc_segment_sum(x, pidx, val):
    half = x.shape[1] // 2
    nck = val.shape[0] // (_NS * _C)
    assert nck % _NB == 0
    np_out = _NE - 624 * _NS

    mesh = plsc.VectorSubcoreMesh(core_axis_name="c", subcore_axis_name="s")

    @functools.partial(
        pl.kernel,
        mesh=mesh,
        out_type=jax.ShapeDtypeStruct((_NC, _NE, half), jnp.float32),
        scratch_types=[
            pltpu.VMEM((_NB, _C), jnp.int32),
            pltpu.VMEM((_NB, _C), jnp.int32),
            pltpu.VMEM((_NB, _C), jnp.int32),
            pltpu.VMEM((_NB, _C), jnp.float32),
            pltpu.VMEM((_NB, _C, half), jnp.float32),
            pltpu.VMEM_SHARED((_NE, half), jnp.float32),
            [pltpu.SemaphoreType.DMA] * _NB,
            [pltpu.SemaphoreType.DMA] * _NB,
            [pltpu.SemaphoreType.DMA] * _NB,
        ],
    )
    def seg_sum(x_ref, pidx_ref, val_ref, out_ref,
                pbuf, rowbuf, colbuf, valbuf, stage, acc,
                gsem, asem, isem):
        cid = lax.axis_index("c")
        sid = lax.axis_index("s")
        col0 = pl.multiple_of(cid * half, 128)
        e0 = pl.multiple_of(sid * nck * _C, 8)
        rows0 = pl.multiple_of(sid * 624 + 8 * jnp.minimum(sid, 2), 8)
        has_extra = sid < np_out // 8

        def fire_idx(k, b):
            off = pl.multiple_of(e0 + k * _C, 8)
            pltpu.async_copy(pidx_ref.at[pl.ds(off, _C)], pbuf.at[b],
                             isem[b])
            pltpu.async_copy(val_ref.at[pl.ds(off, _C)], valbuf.at[b],
                             isem[b])

        def wait_idx(k, b):
            off = pl.multiple_of(e0 + k * _C, 8)
            pltpu.make_async_copy(pidx_ref.at[pl.ds(off, _C)], pbuf.at[b],
                                  isem[b]).wait()
            pltpu.make_async_copy(val_ref.at[pl.ds(off, _C)], valbuf.at[b],
                                  isem[b]).wait()

        def unpack(b):
            for j in range(_C // _L):
                v = pbuf[b, pl.ds(j * _L, _L)]
                colbuf[b, pl.ds(j * _L, _L)] = v & ((1 << _PB) - 1)
                rowbuf[b, pl.ds(j * _L, _L)] = lax.shift_right_logical(
                    v, _PB)

        def fire_gather(b):
            pltpu.async_copy(x_ref.at[colbuf.at[b], pl.ds(col0, half)],
                             stage.at[b], gsem[b])

        def wait_gather(b):
            pltpu.make_async_copy(x_ref.at[colbuf.at[b], pl.ds(col0, half)],
                                  stage.at[b], gsem[b]).wait()

        def scale(b):
            def scale_group(g, carry):
                vv16 = valbuf[b, pl.ds(g * _L, _L)]
                for l in range(_L):
                    i = g * _L + l
                    vs = vv16[l]
                    for j in range(half // _L):
                        stage[b, i, pl.ds(j * _L, _L)] = (
                            stage[b, i, pl.ds(j * _L, _L)] * vs)
                return carry

            lax.fori_loop(0, _C // _L, scale_group, 0)

        def fire_scatter(b):
            pltpu.async_copy(stage.at[b], acc.at[rowbuf.at[b]], asem[b],
                             add=True)

        def wait_scatter(b):
            pltpu.make_async_copy(stage.at[b], acc.at[rowbuf.at[b]],
                                  asem[b]).wait()

        fire_idx(0, 0)
        fire_idx(1, 1)

        zv = jnp.zeros((_L,), jnp.float32)

        def zero_row(i, carry):
            for j in range(half // _L):
                stage[2, i, pl.ds(j * _L, _L)] = zv
            return carry

        lax.fori_loop(0, _C, zero_row, 0)
        for m in range(624 // _C):
            pltpu.sync_copy(stage.at[2, pl.ds(0, _C)],
                            acc.at[pl.ds(rows0 + m * _C, _C)])
        tail = 624 - (624 // _C) * _C
        if tail:
            pltpu.sync_copy(stage.at[2, pl.ds(0, tail)],
                            acc.at[pl.ds(rows0 + 624 - tail, tail)])

        @pl.when(has_extra)
        def _():
            pltpu.sync_copy(stage.at[2, pl.ds(0, 8)],
                            acc.at[pl.ds(rows0 + 624, 8)])

        wait_idx(0, 0)
        unpack(0)
        fire_gather(0)
        plsc.subcore_barrier()

        def step_body(p, carry):
            for u in range(_NB):
                k = p * _NB + u
                b = u
                nb = (u + 1) % _NB
                if u == _NB - 1:
                    wait_scatter(nb)
                else:
                    @pl.when(k >= 2)
                    def _():
                        wait_scatter(nb)
                if u == _NB - 1:
                    @pl.when(p < nck // _NB - 1)
                    def _():
                        wait_idx(k + 1, nb)
                        unpack(nb)
                        fire_gather(nb)
                        fire_idx(k + 2, (u + 2) % _NB)
                else:
                    wait_idx(k + 1, nb)
                    unpack(nb)
                    fire_gather(nb)
                    if u == _NB - 2:
                        @pl.when(p < nck // _NB - 1)
                        def _():
                            fire_idx(k + 2, (u + 2) % _NB)
                    else:
                        fire_idx(k + 2, (u + 2) % _NB)
                wait_gather(b)
                scale(b)
                fire_scatter(b)
            return carry

        lax.fori_loop(0, nck // _NB, step_body, 0)
        for j in range(1, _NB):
            wait_scatter((nck - j) % _NB)

        plsc.subcore_barrier()
        pltpu.sync_copy(acc.at[pl.ds(rows0, 624)],
                        out_ref.at[cid, pl.ds(rows0, 624)])

        @pl.when(has_extra)
        def _():
            r1 = pl.multiple_of(rows0 + 624, 8)
            pltpu.sync_copy(acc.at[pl.ds(r1, 8)],
                            out_ref.at[cid, pl.ds(r1, 8)])

    return seg_sum(x, pidx, val)


def _tc_dense(x, xh2, w_node, b_node, w_edge, b_edge):
    n = x.shape[0]
    d = x.shape[1]
    h = w_node.shape[1]
    half = xh2.shape[2]
    R = 1000
    grid = (n // R,)

    def body(x_ref, xh_ref, wn_ref, bn_ref, we_ref, be_ref, on_ref, oe_ref):
        hn = jnp.dot(x_ref[...], wn_ref[...],
                     preferred_element_type=jnp.float32)
        on_ref[...] = jnp.maximum(hn + bn_ref[...], 0.0)
        we = we_ref[...]
        he = (jnp.dot(xh_ref[0], we[:half],
                      preferred_element_type=jnp.float32)
              + jnp.dot(xh_ref[1], we[half:],
                        preferred_element_type=jnp.float32))
        oe_ref[...] = jnp.maximum(he + be_ref[...], 0.0)

    return pl.pallas_call(
        body,
        grid=grid,
        in_specs=[
            pl.BlockSpec((R, d), lambda i: (i, 0)),
            pl.BlockSpec((2, R, half), lambda i: (0, i, 0)),
            pl.BlockSpec((d, h), lambda i: (0, 0)),
            pl.BlockSpec((1, h), lambda i: (0, 0)),
            pl.BlockSpec((d, h), lambda i: (0, 0)),
            pl.BlockSpec((1, h), lambda i: (0, 0)),
        ],
        out_specs=[
            pl.BlockSpec((R, h), lambda i: (i, 0)),
            pl.BlockSpec((R, h), lambda i: (i, 0)),
        ],
        out_shape=[
            jax.ShapeDtypeStruct((n, h), jnp.float32),
            jax.ShapeDtypeStruct((_NE, h), jnp.float32),
        ],
    )(x, xh2, w_node, b_node, w_edge, b_edge)


def kernel(x, incidence_indices, incidence_values, y, batch_0,
           W_node, b_node, W_edge, b_edge):
    e = incidence_values.shape[0]
    row = incidence_indices[0].astype(jnp.int32)
    col = incidence_indices[1].astype(jnp.int32)
    pidx = (row << _PB) | col
    val = incidence_values
    quantum = _C * _NS * _NB
    e_pad = -(-e // quantum) * quantum
    if e_pad != e:
        pad = e_pad - e
        pidx = jnp.concatenate([pidx, jnp.zeros((pad,), jnp.int32)])
        val = jnp.concatenate([val, jnp.zeros((pad,), val.dtype)])
    xh2 = _sc_segment_sum(x, pidx, val)
    xn, xe = _tc_dense(x, xh2, W_node, b_node.reshape(1, -1),
                       W_edge, b_edge.reshape(1, -1))
    return (y, batch_0, xn, xe)


# speedup vs baseline: 1.1182x; 1.1182x over previous
import functools

import jax
import jax.numpy as jnp
from jax import lax
from jax.experimental import pallas as pl
from jax.experimental.pallas import tpu as pltpu
from jax.experimental.pallas import tpu_sc as plsc

_NE = 10000
_NC = 2
_NS = 16
_L = 16
_C = 128


def _sc_segment_sum(x, row, col, val):
    half = x.shape[1] // 2
    e_total = val.shape[0]
    nck_total = e_total // _C
    nck = nck_total // _NS
    n_extra = nck_total - nck * _NS
    assert nck % 6 == 0
    zr = 104

    mesh = plsc.VectorSubcoreMesh(core_axis_name="c", subcore_axis_name="s")

    @functools.partial(
        pl.kernel,
        mesh=mesh,
        out_type=jax.ShapeDtypeStruct((_NC, _NE, half), jnp.float32),
        scratch_types=[
            pltpu.VMEM((_C,), jnp.int32),
            pltpu.VMEM((_C,), jnp.int32),
            pltpu.VMEM((_C,), jnp.int32),
            pltpu.VMEM((_C,), jnp.int32),
            pltpu.VMEM((_C,), jnp.int32),
            pltpu.VMEM((_C,), jnp.int32),
            pltpu.VMEM((_C,), jnp.float32),
            pltpu.VMEM((_C,), jnp.float32),
            pltpu.VMEM((_C,), jnp.float32),
            pltpu.VMEM((_C, half), jnp.float32),
            pltpu.VMEM((_C, half), jnp.float32),
            pltpu.VMEM((zr, half), jnp.float32),
            pltpu.VMEM_SHARED((_NE, half), jnp.float32),
            [pltpu.SemaphoreType.DMA] * 2,
            [pltpu.SemaphoreType.DMA] * 2,
            [pltpu.SemaphoreType.DMA] * 3,
        ],
    )
    def seg_sum(x_ref, row_ref, col_ref, val_ref, out_ref,
                rowb0, rowb1, rowb2, colb0, colb1, colb2,
                valb0, valb1, valb2, st0, st1,
                zbuf, acc, gsem, asem, isem):
        cid = lax.axis_index("c")
        sid = lax.axis_index("s")
        col0 = pl.multiple_of(cid * half, 128)
        ck0 = sid * nck
        rows0 = pl.multiple_of(sid * 624 + 8 * jnp.minimum(sid, 2), 8)
        has_extra = sid < n_extra

        stages = [st0, st1]
        idxsets = [(rowb0, colb0, valb0), (rowb1, colb1, valb1),
                   (rowb2, colb2, valb2)]

        def fire_idx(ck, i3):
            rowb, colb, valb = idxsets[i3]
            off = pl.multiple_of(ck * _C, 8)
            pltpu.async_copy(row_ref.at[pl.ds(off, _C)], rowb, isem[i3])
            pltpu.async_copy(col_ref.at[pl.ds(off, _C)], colb, isem[i3])
            pltpu.async_copy(val_ref.at[pl.ds(off, _C)], valb, isem[i3])

        def wait_idx(ck, i3):
            rowb, colb, valb = idxsets[i3]
            off = pl.multiple_of(ck * _C, 8)
            pltpu.make_async_copy(row_ref.at[pl.ds(off, _C)], rowb,
                                  isem[i3]).wait()
            pltpu.make_async_copy(col_ref.at[pl.ds(off, _C)], colb,
                                  isem[i3]).wait()
            pltpu.make_async_copy(val_ref.at[pl.ds(off, _C)], valb,
                                  isem[i3]).wait()

        def fire_gather(i2, i3):
            colb = idxsets[i3][1]
            pltpu.async_copy(x_ref.at[colb, pl.ds(col0, half)],
                             stages[i2], gsem[i2])

        def wait_gather(i2, i3):
            colb = idxsets[i3][1]
            pltpu.make_async_copy(x_ref.at[colb, pl.ds(col0, half)],
                                  stages[i2], gsem[i2]).wait()

        def scale(i2, i3):
            valb = idxsets[i3][2]
            st = stages[i2]

            def scale_group(g, carry):
                vv16 = valb[pl.ds(g * _L, _L)]
                for l in range(_L):
                    i = g * _L + l
                    vs = vv16[l]
                    for j in range(half // _L):
                        st[i, pl.ds(j * _L, _L)] = (
                            st[i, pl.ds(j * _L, _L)] * vs)
                return carry

            lax.fori_loop(0, _C // _L, scale_group, 0)

        def fire_scatter(i2, i3):
            rowb = idxsets[i3][0]
            pltpu.async_copy(stages[i2], acc.at[rowb], asem[i2], add=True)

        def wait_scatter(i2, i3):
            rowb = idxsets[i3][0]
            pltpu.make_async_copy(stages[i2], acc.at[rowb],
                                  asem[i2]).wait()

        fire_idx(ck0, 0)
        fire_idx(ck0 + 1, 1)

        zv = jnp.zeros((_L,), jnp.float32)

        def zero_row(i, carry):
            for j in range(half // _L):
                zbuf[i, pl.ds(j * _L, _L)] = zv
            return carry

        lax.fori_loop(0, zr, zero_row, 0)
        for k in range(624 // zr):
            pltpu.sync_copy(zbuf, acc.at[pl.ds(rows0 + k * zr, zr)])

        @pl.when(has_extra)
        def _():
            pltpu.sync_copy(zbuf.at[pl.ds(0, 8)],
                            acc.at[pl.ds(rows0 + 624, 8)])

        wait_idx(ck0, 0)
        fire_gather(0, 0)
        plsc.subcore_barrier()

        def step_body(p, carry):
            for u in range(6):
                j = p * 6 + u
                s2, s3 = u % 2, u % 3
                n2, n3 = (u + 1) % 2, (u + 1) % 3
                if u == 0:
                    @pl.when(j >= 1)
                    def _():
                        wait_scatter(n2, (u - 1) % 3)
                else:
                    wait_scatter(n2, (u - 1) % 3)
                if u in (4, 5):
                    @pl.when(p < nck // 6 - 1)
                    def _():
                        fire_idx(ck0 + j + 2, (u + 2) % 3)
                else:
                    fire_idx(ck0 + j + 2, (u + 2) % 3)
                if u == 5:
                    @pl.when(p < nck // 6 - 1)
                    def _():
                        wait_idx(ck0 + j + 1, n3)
                        fire_gather(n2, n3)
                else:
                    wait_idx(ck0 + j + 1, n3)
                    fire_gather(n2, n3)
                wait_gather(s2, s3)
                scale(s2, s3)
                fire_scatter(s2, s3)
            return carry

        lax.fori_loop(0, nck // 6, step_body, 0)
        wait_scatter((nck - 1) % 2, (nck - 1) % 3)

        @pl.when(has_extra)
        def _():
            ck = nck * _NS + sid
            fire_idx(ck, 0)
            wait_idx(ck, 0)
            fire_gather(0, 0)
            wait_gather(0, 0)
            scale(0, 0)
            fire_scatter(0, 0)
            wait_scatter(0, 0)

        plsc.subcore_barrier()
        pltpu.sync_copy(acc.at[pl.ds(rows0, 624)],
                        out_ref.at[cid, pl.ds(rows0, 624)])

        @pl.when(has_extra)
        def _():
            r1 = pl.multiple_of(rows0 + 624, 8)
            pltpu.sync_copy(acc.at[pl.ds(r1, 8)],
                            out_ref.at[cid, pl.ds(r1, 8)])

    return seg_sum(x, row, col, val)


def _tc_dense(x, xh2, w_node, b_node, w_edge, b_edge):
    n = x.shape[0]
    d = x.shape[1]
    h = w_node.shape[1]
    half = xh2.shape[2]
    R = 1000
    grid = (n // R,)

    def body(x_ref, xh_ref, wn_ref, bn_ref, we_ref, be_ref, on_ref, oe_ref):
        hn = jnp.dot(x_ref[...], wn_ref[...],
                     preferred_element_type=jnp.float32)
        on_ref[...] = jnp.maximum(hn + bn_ref[...], 0.0)
        we = we_ref[...]
        he = (jnp.dot(xh_ref[0], we[:half],
                      preferred_element_type=jnp.float32)
              + jnp.dot(xh_ref[1], we[half:],
                        preferred_element_type=jnp.float32))
        oe_ref[...] = jnp.maximum(he + be_ref[...], 0.0)

    return pl.pallas_call(
        body,
        grid=grid,
        in_specs=[
            pl.BlockSpec((R, d), lambda i: (i, 0)),
            pl.BlockSpec((2, R, half), lambda i: (0, i, 0)),
            pl.BlockSpec((d, h), lambda i: (0, 0)),
            pl.BlockSpec((1, h), lambda i: (0, 0)),
            pl.BlockSpec((d, h), lambda i: (0, 0)),
            pl.BlockSpec((1, h), lambda i: (0, 0)),
        ],
        out_specs=[
            pl.BlockSpec((R, h), lambda i: (i, 0)),
            pl.BlockSpec((R, h), lambda i: (i, 0)),
        ],
        out_shape=[
            jax.ShapeDtypeStruct((n, h), jnp.float32),
            jax.ShapeDtypeStruct((_NE, h), jnp.float32),
        ],
    )(x, xh2, w_node, b_node, w_edge, b_edge)


def kernel(x, incidence_indices, incidence_values, y, batch_0,
           W_node, b_node, W_edge, b_edge):
    row = incidence_indices[0].astype(jnp.int32)
    col = incidence_indices[1].astype(jnp.int32)
    xh2 = _sc_segment_sum(x, row, col, incidence_values)
    xn, xe = _tc_dense(x, xh2, W_node, b_node.reshape(1, -1),
                       W_edge, b_edge.reshape(1, -1))
    return (y, batch_0, xn, xe)
